# baseline (speedup 1.0000x reference)
import jax
import jax.numpy as jnp
from jax import lax
from jax.experimental import pallas as pl
from jax.experimental.pallas import tpu as pltpu
from jax.experimental.pallas import tpu_sc as plsc

_B, _S = 4096, 200
_D = 64
_L = 16

_NC, _NS = 2, 16
_NW = _NC * _NS
_BBLK = _B // _NW
_CH = 2
_NCH = _S // _CH
_NBUF = 2
assert (_NCH - 2 * _NBUF) % _NBUF == 0


def _gather_body(table_hbm, idx_hbm, out_hbm, idx_v, *scratch):
    gbufs = scratch[:_NBUF]
    tbufs = scratch[_NBUF:2 * _NBUF]
    gsems = scratch[2 * _NBUF:3 * _NBUF]
    wsems = scratch[3 * _NBUF:4 * _NBUF]

    wid = lax.axis_index("s") * _NC + lax.axis_index("c")
    col0 = wid * _BBLK

    pltpu.sync_copy(idx_hbm.at[:, pl.ds(col0, _BBLK)], idx_v)

    lane = jnp.arange(_L, dtype=jnp.int32)
    zeros = jnp.zeros((_L,), jnp.int32)
    skews = [(lane + c) & (_L - 1) for c in range(_L)]
    ls_base = [lane * _D + skews[c] for c in range(_L)]
    ss_base = [skews[c] * _BBLK + lane for c in range(_L)]

    def fire_gather(ch, b):
        for i in range(_CH):
            pltpu.async_copy(
                table_hbm.at[idx_v.at[ch * _CH + i]],
                gbufs[b].at[pl.ds(i * _BBLK, _BBLK)], gsems[b])

    def wait_gather(ch, b):
        for i in range(_CH):
            pltpu.make_async_copy(
                table_hbm.at[idx_v.at[ch * _CH + i]],
                gbufs[b].at[pl.ds(i * _BBLK, _BBLK)], gsems[b]).wait()

    def transpose(b):
        g, t = gbufs[b], tbufs[b]

        @pl.loop(0, _D, step=_L)
        def _dloop(d0):
            for i in range(_CH):
                for j0 in range(_BBLK // _L):
                    lo = zeros + (i * _BBLK * _D + d0 + j0 * (_L * _D))
                    so = zeros + (i * _BBLK * _D + d0 * _BBLK + j0 * _L)
                    for c0 in range(0, _L, 8):
                        vals = [
                            plsc.load_gather(g, [zeros, ls_base[c0 + k] + lo])
                            for k in range(8)]
                        for k in range(8):
                            plsc.store_scatter(
                                t, [zeros, zeros, ss_base[c0 + k] + so],
                                vals[k])

    def fire_write(ch, b):
        for i in range(_CH):
            pltpu.async_copy(
                tbufs[b].at[i],
                out_hbm.at[ch * _CH + i, :, pl.ds(col0, _BBLK)], wsems[b]
            )

    def wait_write(ch, b):
        for i in range(_CH):
            pltpu.make_async_copy(
                tbufs[b].at[i],
                out_hbm.at[ch * _CH + i, :, pl.ds(col0, _BBLK)], wsems[b]
            ).wait()

    for b in range(_NBUF):
        fire_gather(b, b)
    for c in range(_NBUF):
        b = c % _NBUF
        wait_gather(c, b)
        transpose(b)
        fire_gather(c + _NBUF, b)
        fire_write(c, b)

    @pl.loop(_NBUF, _NCH - _NBUF, step=_NBUF)
    def _steady(c0):
        for b in range(_NBUF):
            c = c0 + b
            wait_gather(c, b)
            wait_write(c - _NBUF, b)
            transpose(b)
            fire_gather(c + _NBUF, b)
            fire_write(c, b)

    for c in range(_NCH - _NBUF, _NCH):
        b = c % _NBUF
        wait_gather(c, b)
        wait_write(c - _NBUF, b)
        transpose(b)
        fire_write(c, b)
    for c in range(_NCH - _NBUF, _NCH):
        wait_write(c, c % _NBUF)


@jax.jit
def _sc_gather(table, idx_t):
    mesh = plsc.VectorSubcoreMesh(core_axis_name="c", subcore_axis_name="s")
    run = pl.kernel(
        _gather_body,
        out_type=jax.ShapeDtypeStruct((_S, _D, _B), jnp.float32),
        mesh=mesh,
        scratch_types=(
            [pltpu.VMEM((_S, _BBLK), jnp.int32)]
            + [pltpu.VMEM((_CH * _BBLK, _D), jnp.float32)] * _NBUF
            + [pltpu.VMEM((_CH, _D, _BBLK), jnp.float32)] * _NBUF
            + [pltpu.SemaphoreType.DMA] * (2 * _NBUF)
        ),
        compiler_params=pltpu.CompilerParams(
            use_tc_tiling_on_sc=False,
            needs_layout_passes=False,
            disable_bounds_checks=True,
        ),
    )
    return run(table, idx_t)


def kernel(x, table):
    idx_t = jnp.transpose(x)
    out_t = _sc_gather(table, idx_t)
    return jnp.transpose(out_t, (2, 0, 1))


# speedup vs baseline: 1.5741x; 1.5741x over previous
import jax
import jax.numpy as jnp
from jax import lax
from jax.experimental import pallas as pl
from jax.experimental.pallas import tpu as pltpu
from jax.experimental.pallas import tpu_sc as plsc

_B, _S = 4096, 200
_D = 64
_L = 16

_NC, _NS = 2, 16
_NW = _NC * _NS
_BBLK = _B // _NW
_CH = 2
_NCH = _S // _CH
_NBUF = 2
_GW = 128
assert (_NCH - 2 * _NBUF) % _NBUF == 0


def _gather_body(table_hbm, idx_hbm, out_hbm, idx_v, *scratch):
    gbufs = scratch[:_NBUF]
    tbufs = scratch[_NBUF:2 * _NBUF]
    gsems = scratch[2 * _NBUF:3 * _NBUF]
    wsems = scratch[3 * _NBUF:4 * _NBUF]

    wid = lax.axis_index("s") * _NC + lax.axis_index("c")
    col0 = wid * _BBLK

    pltpu.sync_copy(idx_hbm.at[:, pl.ds(col0, _BBLK)], idx_v)

    lane = jnp.arange(_L, dtype=jnp.int32)
    zeros = jnp.zeros((_L,), jnp.int32)
    skews = [(lane + c) & (_L - 1) for c in range(_L)]
    ls_base = [lane * _GW + skews[c] for c in range(_L)]
    ss_base = [skews[c] * _BBLK + lane for c in range(_L)]

    def fire_gather(ch, b):
        for i in range(_CH):
            pltpu.async_copy(
                table_hbm.at[idx_v.at[ch * _CH + i]],
                gbufs[b].at[pl.ds(i * _BBLK, _BBLK)], gsems[b])

    def wait_gather(ch, b):
        for i in range(_CH):
            pltpu.make_async_copy(
                table_hbm.at[idx_v.at[ch * _CH + i]],
                gbufs[b].at[pl.ds(i * _BBLK, _BBLK)], gsems[b]).wait()

    def transpose(b):
        g, t = gbufs[b], tbufs[b]

        @pl.loop(0, _D, step=_L)
        def _dloop(d0):
            for i in range(_CH):
                for j0 in range(_BBLK // _L):
                    lo = zeros + (i * _BBLK * _GW + d0 + j0 * (_L * _GW))
                    so = zeros + (i * _BBLK * _D + d0 * _BBLK + j0 * _L)
                    for c0 in range(0, _L, 8):
                        vals = [
                            plsc.load_gather(g, [zeros, ls_base[c0 + k] + lo])
                            for k in range(8)]
                        for k in range(8):
                            plsc.store_scatter(
                                t, [zeros, zeros, ss_base[c0 + k] + so],
                                vals[k])

    def fire_write(ch, b):
        for i in range(_CH):
            pltpu.async_copy(
                tbufs[b].at[i],
                out_hbm.at[ch * _CH + i, :, pl.ds(col0, _BBLK)], wsems[b]
            )

    def wait_write(ch, b):
        for i in range(_CH):
            pltpu.make_async_copy(
                tbufs[b].at[i],
                out_hbm.at[ch * _CH + i, :, pl.ds(col0, _BBLK)], wsems[b]
            ).wait()

    for b in range(_NBUF):
        fire_gather(b, b)
    for c in range(_NBUF):
        b = c % _NBUF
        wait_gather(c, b)
        transpose(b)
        fire_gather(c + _NBUF, b)
        fire_write(c, b)

    @pl.loop(_NBUF, _NCH - _NBUF, step=_NBUF)
    def _steady(c0):
        for b in range(_NBUF):
            c = c0 + b
            wait_gather(c, b)
            wait_write(c - _NBUF, b)
            transpose(b)
            fire_gather(c + _NBUF, b)
            fire_write(c, b)

    for c in range(_NCH - _NBUF, _NCH):
        b = c % _NBUF
        wait_gather(c, b)
        wait_write(c - _NBUF, b)
        transpose(b)
        fire_write(c, b)
    for c in range(_NCH - _NBUF, _NCH):
        wait_write(c, c % _NBUF)


@jax.jit
def _sc_gather(table, idx_t):
    mesh = plsc.VectorSubcoreMesh(core_axis_name="c", subcore_axis_name="s")
    run = pl.kernel(
        _gather_body,
        out_type=jax.ShapeDtypeStruct((_S, _D, _B), jnp.float32),
        mesh=mesh,
        scratch_types=(
            [pltpu.VMEM((_S, _BBLK), jnp.int32)]
            + [pltpu.VMEM((_CH * _BBLK, _GW), jnp.float32)] * _NBUF
            + [pltpu.VMEM((_CH, _D, _BBLK), jnp.float32)] * _NBUF
            + [pltpu.SemaphoreType.DMA] * (2 * _NBUF)
        ),
        compiler_params=pltpu.CompilerParams(
            use_tc_tiling_on_sc=True,
            needs_layout_passes=False,
            disable_bounds_checks=True,
        ),
    )
    return run(table, idx_t)


def kernel(x, table):
    idx_t = jnp.transpose(x)
    table_pad = jnp.pad(table, ((0, 0), (0, _GW - _D)))
    out_t = _sc_gather(table_pad, idx_t)
    return jnp.transpose(out_t, (2, 0, 1))
